# baseline (speedup 1.0000x reference)
import functools

import jax
import jax.numpy as jnp
from jax import lax
from jax.experimental import pallas as pl
from jax.experimental.pallas import tpu as pltpu
from jax.experimental.pallas import tpu_sc as plsc

_NE = 8192
_ED = 256
_NTOK = 16384
_BETA = 0.25
_BN = 256
_NB = _NTOK // _BN


_GW = 2048


def _bf16(x):
    return x.astype(jnp.bfloat16).astype(jnp.float32)


def _tc_body(zb_ref, zsq_ref, embt_ref, esq_ref, erow_ref,
             idx_ref, part_ref):
    zb = zb_ref[...]
    ez = jnp.dot(zb, embt_ref[...], preferred_element_type=jnp.float32)
    dists = (zsq_ref[...] + esq_ref[...]) - 2.0 * ez
    lane = lax.broadcasted_iota(jnp.int32, (_BN, _NE), 1)
    acc_v = None
    acc_g = None
    for g in range(_NE // _GW):
        lo = g * _GW
        hi = min(lo + _GW, _NE)
        mask = (lane >= lo) & (lane < hi)
        mg = jnp.min(jnp.where(mask, dists, jnp.inf), axis=1)
        if g == 0:
            acc_v = mg
            acc_g = jnp.zeros_like(mg, dtype=jnp.int32)
        else:
            upd = mg < _bf16(acc_v)
            acc_v = jnp.where(upd, mg, acc_v)
            acc_g = jnp.where(upd, jnp.int32(g), acc_g)
    glo = acc_g[:, None] * _GW
    in_g = (lane >= glo) & (lane < glo + _GW)
    hit = in_g & (dists == acc_v[:, None])
    idx = jnp.min(jnp.where(hit, lane, _NE), axis=1)
    idx_ref[0, 0, :] = idx
    er_sel = jnp.sum(jnp.where(lane == idx[:, None], erow_ref[...], 0.0), axis=1)
    s0 = jnp.sum(acc_v)
    s1 = jnp.sum(er_sel)
    s2 = jnp.sum(zb)
    li = lax.broadcasted_iota(jnp.int32, (1, 128), 1)
    part = jnp.where(li == 0, s0, jnp.where(li == 1, s1,
                     jnp.where(li == 2, s2, 0.0)))
    part_ref[0, :, :] = part


_tc_call = pl.pallas_call(
    _tc_body,
    grid=(_NB,),
    in_specs=[
        pl.BlockSpec((_BN, _ED), lambda i: (i, 0)),
        pl.BlockSpec((_BN, 1), lambda i: (i, 0)),
        pl.BlockSpec((_ED, _NE), lambda i: (0, 0)),
        pl.BlockSpec((1, _NE), lambda i: (0, 0)),
        pl.BlockSpec((1, _NE), lambda i: (0, 0)),
    ],
    out_specs=[
        pl.BlockSpec((1, 1, _BN), lambda i: (i, 0, 0)),
        pl.BlockSpec((1, 1, 128), lambda i: (i, 0, 0)),
    ],
    out_shape=[
        jax.ShapeDtypeStruct((_NB, 1, _BN), jnp.int32),
        jax.ShapeDtypeStruct((_NB, 1, 128), jnp.float32),
    ],
)


def _make_sc_gather():
    info = plsc.get_sparse_core_info()
    nc, ns = info.num_cores, info.num_subcores
    nw = nc * ns
    bpw = _NTOK // nw
    ch = 128
    nch = bpw // ch
    mesh = plsc.VectorSubcoreMesh(core_axis_name="c", subcore_axis_name="s")

    @functools.partial(
        pl.kernel, mesh=mesh,
        out_type=jax.ShapeDtypeStruct((_NTOK, _ED), jnp.float32),
        scratch_types=[
            pltpu.VMEM((ch,), jnp.int32),
            pltpu.VMEM((ch, _ED), jnp.float32),
            pltpu.SemaphoreType.DMA,
        ],
    )
    def gk(idx_hbm, table_hbm, out_hbm, idx_v, rows_v, sem):
        wid = lax.axis_index("s") * nc + lax.axis_index("c")
        base = wid * bpw
        for c in range(nch):
            off = base + c * ch
            pltpu.sync_copy(idx_hbm.at[pl.ds(off, ch)], idx_v)
            pltpu.async_copy(table_hbm.at[idx_v], rows_v, sem).wait()
            pltpu.sync_copy(rows_v, out_hbm.at[pl.ds(off, ch)])

    return gk


def kernel(z, emb):
    zp = jnp.transpose(z, (0, 2, 3, 1))
    z_flat = zp.reshape(-1, _ED)
    z_sq = jnp.sum(z_flat ** 2, axis=1, keepdims=True)
    e_sq = jnp.sum(emb ** 2, axis=1)
    e_row = jnp.sum(emb, axis=1)
    embt = emb.T

    idx3, parts = _tc_call(z_flat, z_sq, embt, e_sq[None, :], e_row[None, :])
    idx = idx3.reshape(-1)

    z_q_flat = _make_sc_gather()(idx, emb)

    sums = jnp.sum(parts[:, 0, :], axis=0)
    n = float(_NTOK * _ED)
    embedding_loss = sums[0] / n
    commitment_loss = _BETA * (sums[1] - sums[2]) / n
    loss = embedding_loss + commitment_loss

    z_q_out = jnp.transpose(z_q_flat.reshape(zp.shape), (0, 3, 1, 2))
    return (z_q_out, loss)


# speedup vs baseline: 1.0010x; 1.0010x over previous
import functools

import jax
import jax.numpy as jnp
from jax import lax
from jax.experimental import pallas as pl
from jax.experimental.pallas import tpu as pltpu
from jax.experimental.pallas import tpu_sc as plsc

_NE = 8192
_ED = 256
_NTOK = 16384
_BETA = 0.25
_BN = 256
_NB = _NTOK // _BN


_GW = 2048


def _bf16(x):
    return x.astype(jnp.bfloat16).astype(jnp.float32)


def _tc_body(zb_ref, zsq_ref, embt_ref, esq_ref, erow_ref,
             idx_ref, part_ref):
    zb = zb_ref[...]
    ez = jnp.dot(zb, embt_ref[...], preferred_element_type=jnp.float32)
    dists = (zsq_ref[...] + esq_ref[...]) - 2.0 * ez
    lane = lax.broadcasted_iota(jnp.int32, (_BN, _NE), 1)
    acc_v = None
    acc_g = None
    for g in range(_NE // _GW):
        lo = g * _GW
        mg = jnp.min(dists[:, lo:lo + _GW], axis=1)
        if g == 0:
            acc_v = mg
            acc_g = jnp.zeros_like(mg, dtype=jnp.int32)
        else:
            upd = mg < _bf16(acc_v)
            acc_v = jnp.where(upd, mg, acc_v)
            acc_g = jnp.where(upd, jnp.int32(g), acc_g)
    glo = acc_g[:, None] * _GW
    in_g = (lane >= glo) & (lane < glo + _GW)
    hit = in_g & (dists == acc_v[:, None])
    idx = jnp.min(jnp.where(hit, lane, _NE), axis=1)
    idx_ref[0, 0, :] = idx
    er_sel = jnp.sum(jnp.where(lane == idx[:, None], erow_ref[...], 0.0), axis=1)
    s0 = jnp.sum(acc_v)
    s1 = jnp.sum(er_sel)
    s2 = jnp.sum(zb)
    li = lax.broadcasted_iota(jnp.int32, (1, 128), 1)
    part = jnp.where(li == 0, s0, jnp.where(li == 1, s1,
                     jnp.where(li == 2, s2, 0.0)))
    part_ref[0, :, :] = part


_tc_call = pl.pallas_call(
    _tc_body,
    grid=(_NB,),
    in_specs=[
        pl.BlockSpec((_BN, _ED), lambda i: (i, 0)),
        pl.BlockSpec((_BN, 1), lambda i: (i, 0)),
        pl.BlockSpec((_ED, _NE), lambda i: (0, 0)),
        pl.BlockSpec((1, _NE), lambda i: (0, 0)),
        pl.BlockSpec((1, _NE), lambda i: (0, 0)),
    ],
    out_specs=[
        pl.BlockSpec((1, 1, _BN), lambda i: (i, 0, 0)),
        pl.BlockSpec((1, 1, 128), lambda i: (i, 0, 0)),
    ],
    out_shape=[
        jax.ShapeDtypeStruct((_NB, 1, _BN), jnp.int32),
        jax.ShapeDtypeStruct((_NB, 1, 128), jnp.float32),
    ],
)


def _make_sc_gather():
    info = plsc.get_sparse_core_info()
    nc, ns = info.num_cores, info.num_subcores
    nw = nc * ns
    bpw = _NTOK // nw
    ch = 128
    nch = bpw // ch
    mesh = plsc.VectorSubcoreMesh(core_axis_name="c", subcore_axis_name="s")

    @functools.partial(
        pl.kernel, mesh=mesh,
        out_type=jax.ShapeDtypeStruct((_NTOK, _ED), jnp.float32),
        scratch_types=[
            pltpu.VMEM((ch,), jnp.int32),
            pltpu.VMEM((ch, _ED), jnp.float32),
            pltpu.SemaphoreType.DMA,
        ],
    )
    def gk(idx_hbm, table_hbm, out_hbm, idx_v, rows_v, sem):
        wid = lax.axis_index("s") * nc + lax.axis_index("c")
        base = wid * bpw
        for c in range(nch):
            off = base + c * ch
            pltpu.sync_copy(idx_hbm.at[pl.ds(off, ch)], idx_v)
            pltpu.async_copy(table_hbm.at[idx_v], rows_v, sem).wait()
            pltpu.sync_copy(rows_v, out_hbm.at[pl.ds(off, ch)])

    return gk


def kernel(z, emb):
    zp = jnp.transpose(z, (0, 2, 3, 1))
    z_flat = zp.reshape(-1, _ED)
    z_sq = jnp.sum(z_flat ** 2, axis=1, keepdims=True)
    e_sq = jnp.sum(emb ** 2, axis=1)
    e_row = jnp.sum(emb, axis=1)
    embt = emb.T

    idx3, parts = _tc_call(z_flat, z_sq, embt, e_sq[None, :], e_row[None, :])
    idx = idx3.reshape(-1)

    z_q_flat = _make_sc_gather()(idx, emb)

    sums = jnp.sum(parts[:, 0, :], axis=0)
    n = float(_NTOK * _ED)
    embedding_loss = sums[0] / n
    commitment_loss = _BETA * (sums[1] - sums[2]) / n
    loss = embedding_loss + commitment_loss

    z_q_out = jnp.transpose(z_q_flat.reshape(zp.shape), (0, 3, 1, 2))
    return (z_q_out, loss)
